# baseline (speedup 1.0000x reference)
import functools

import jax
import jax.numpy as jnp
import numpy as np
from jax.experimental import pallas as pl
from jax.experimental.pallas import tpu as pltpu

LAMBDA_DEPTHG = 0.3
TEMP = 0.1
KNN_K = 7
SIGMA_D = 0.5

BLK = 512
CHEB_M = 16


def _cheb_coeff_2d(c, m):
    j = np.arange(m)
    th = np.pi * (j + 0.5) / m
    u = (np.cos(th) + 1.0) / 2.0
    f = np.exp(c * (u[:, None] - u[None, :]) ** 2)
    p = np.cos(np.outer(j, th)) * (2.0 / m)
    p[0] *= 0.5
    return (p @ f @ p.T).astype(np.float32)


_A_W = _cheb_coeff_2d(-1.0 / (2.0 * SIGMA_D * SIGMA_D), CHEB_M)
_A_W2 = _cheb_coeff_2d(-1.0 / (SIGMA_D * SIGMA_D), CHEB_M)


def _body(c_ref, f_ref, dc_ref, a_ref, a2_ref, pos_ref, lse_ref, dsq_ref,
          cn_ref, fn_ref, craw_ref, fraw_ref, csem, fsem, *, n):
    b = pl.program_id(0)
    i = pl.program_id(1)

    @pl.when((b == 0) & (i == 0))
    def _init():
        pos_ref[...] = jnp.zeros((1, 1), jnp.float32)
        lse_ref[...] = jnp.zeros((1, 1), jnp.float32)
        dsq_ref[...] = jnp.zeros((1, 1), jnp.float32)

    @pl.when(i == 0)
    def _per_batch():
        cp_f = pltpu.make_async_copy(f_ref.at[b], fraw_ref, fsem)
        cp_f.start()
        cp_c = pltpu.make_async_copy(c_ref.at[b], craw_ref, csem)
        cp_c.start()
        cp_f.wait()
        cp_c.wait()
        f_all = fraw_ref[...]
        fn_ref[...] = (
            f_all * jax.lax.rsqrt(jnp.sum(f_all * f_all, axis=1, keepdims=True) + 1e-12)
        ).astype(jnp.bfloat16)
        c_all = craw_ref[...]
        cf = c_all * jax.lax.rsqrt(
            jnp.sum(c_all * c_all, axis=1, keepdims=True) + 1e-12
        )
        cn_ref[...] = cf.astype(jnp.bfloat16)

        xt = 2.0 * dc_ref[0] - 1.0
        ts = [jnp.ones_like(xt), xt]
        for _ in range(CHEB_M - 2):
            ts.append(2.0 * xt * ts[-1] - ts[-2])
        phi = jnp.concatenate(ts, axis=0)
        v = jax.lax.dot_general(
            phi, cf, (((1,), (0,)), ((), ())), preferred_element_type=jnp.float32
        )
        vv = jax.lax.dot_general(
            v, v, (((1,), (1,)), ((), ())), preferred_element_type=jnp.float32
        )
        g = jax.lax.dot_general(
            cf, cf, (((0,), (0,)), ((), ())), preferred_element_type=jnp.float32
        )
        s = jnp.sum(phi, axis=1, keepdims=True)
        ss = jax.lax.dot_general(
            s, s, (((1,), (1,)), ((), ())), preferred_element_type=jnp.float32
        )
        dsq_ref[...] += (
            jnp.sum(g * g, axis=(0, 1), keepdims=True)
            - 2.0 * jnp.sum(a_ref[...] * vv, axis=(0, 1), keepdims=True)
            + jnp.sum(a2_ref[...] * ss, axis=(0, 1), keepdims=True)
        )

    row0 = i * BLK
    fn = fn_ref[...]
    cn = cn_ref[...]
    fb = fn_ref[pl.ds(row0, BLK), :]
    cb = cn_ref[pl.ds(row0, BLK), :]

    sim_f = jax.lax.dot_general(
        fb, fn, (((1,), (1,)), ((), ())), preferred_element_type=jnp.float32
    )
    sim_c = jax.lax.dot_general(
        cb, cn, (((1,), (1,)), ((), ())), preferred_element_type=jnp.float32
    )

    cols = jax.lax.broadcasted_iota(jnp.int32, (BLK, n), 1)
    rows = jax.lax.broadcasted_iota(jnp.int32, (BLK, n), 0) + row0
    sim_f = jnp.where(cols == rows, -1e9, sim_f)

    lse = jnp.log(jnp.sum(jnp.exp(sim_c * (1.0 / TEMP)), axis=1, keepdims=True))
    lse_ref[...] += jnp.sum(lse, axis=(0, 1), keepdims=True)

    neg = jnp.float32(-jnp.inf)
    mx = jnp.max(sim_f, axis=1, keepdims=True)
    for _ in range(KNN_K - 1):
        mx = jnp.max(jnp.where(sim_f < mx, sim_f, neg), axis=1, keepdims=True)
    pos_ref[...] += jnp.sum(
        jnp.where(sim_f >= mx, sim_c, 0.0), axis=(0, 1), keepdims=True
    )


def _loss_sums(semantic_codes, dino_features, depth):
    bsz, n, dc = semantic_codes.shape
    df = dino_features.shape[-1]
    depth_c = depth[:, None, :]
    a_w = jnp.asarray(_A_W)
    a_w2 = jnp.asarray(_A_W2)

    grid = (bsz, n // BLK)
    return pl.pallas_call(
        functools.partial(_body, n=n),
        grid=grid,
        in_specs=[
            pl.BlockSpec(memory_space=pl.ANY),
            pl.BlockSpec(memory_space=pl.ANY),
            pl.BlockSpec((1, 1, n), lambda b, i: (b, 0, 0)),
            pl.BlockSpec((CHEB_M, CHEB_M), lambda b, i: (0, 0)),
            pl.BlockSpec((CHEB_M, CHEB_M), lambda b, i: (0, 0)),
        ],
        out_specs=[
            pl.BlockSpec((1, 1), lambda b, i: (0, 0)),
            pl.BlockSpec((1, 1), lambda b, i: (0, 0)),
            pl.BlockSpec((1, 1), lambda b, i: (0, 0)),
        ],
        out_shape=[
            jax.ShapeDtypeStruct((1, 1), jnp.float32),
            jax.ShapeDtypeStruct((1, 1), jnp.float32),
            jax.ShapeDtypeStruct((1, 1), jnp.float32),
        ],
        scratch_shapes=[
            pltpu.VMEM((n, dc), jnp.bfloat16),
            pltpu.VMEM((n, df), jnp.bfloat16),
            pltpu.VMEM((n, dc), jnp.float32),
            pltpu.VMEM((n, df), jnp.float32),
            pltpu.SemaphoreType.DMA,
            pltpu.SemaphoreType.DMA,
        ],
    )(semantic_codes, dino_features, depth_c, a_w, a_w2)


@jax.jit
def kernel(semantic_codes, dino_features, depth):
    bsz, n, _ = semantic_codes.shape
    pos_sum, lse_sum, dsq_sum = _loss_sums(
        semantic_codes, dino_features, depth
    )

    l_stego = -(pos_sum[0, 0] * (1.0 / TEMP) - KNN_K * lse_sum[0, 0]) / (
        bsz * n * KNN_K
    )
    l_depthg = dsq_sum[0, 0] / (bsz * n * n)
    total = l_stego + LAMBDA_DEPTHG * l_depthg
    return (l_stego, l_depthg, total)


# speedup vs baseline: 1.1086x; 1.1086x over previous
import functools

import jax
import jax.numpy as jnp
import numpy as np
from jax.experimental import pallas as pl
from jax.experimental.pallas import tpu as pltpu

LAMBDA_DEPTHG = 0.3
TEMP = 0.1
KNN_K = 7
SIGMA_D = 0.5

BLK = 1024
CHEB_M = 16


def _cheb_coeff_2d(c, m):
    j = np.arange(m)
    th = np.pi * (j + 0.5) / m
    u = (np.cos(th) + 1.0) / 2.0
    f = np.exp(c * (u[:, None] - u[None, :]) ** 2)
    p = np.cos(np.outer(j, th)) * (2.0 / m)
    p[0] *= 0.5
    return (p @ f @ p.T).astype(np.float32)


_A_W = _cheb_coeff_2d(-1.0 / (2.0 * SIGMA_D * SIGMA_D), CHEB_M)
_A_W2 = _cheb_coeff_2d(-1.0 / (SIGMA_D * SIGMA_D), CHEB_M)


def _body(c_ref, f_ref, dc_ref, a_ref, a2_ref, pos_ref, lse_ref, dsq_ref,
          cn_ref, fn_ref, *, n):
    b = pl.program_id(0)
    i = pl.program_id(1)

    @pl.when((b == 0) & (i == 0))
    def _init():
        pos_ref[...] = jnp.zeros((1, 1), jnp.float32)
        lse_ref[...] = jnp.zeros((1, 1), jnp.float32)
        dsq_ref[...] = jnp.zeros((1, 1), jnp.float32)

    @pl.when(i == 0)
    def _per_batch():
        f_all = f_ref[0]
        fn_ref[...] = (
            f_all * jax.lax.rsqrt(jnp.sum(f_all * f_all, axis=1, keepdims=True) + 1e-12)
        ).astype(jnp.bfloat16)
        c_all = c_ref[0]
        cf = c_all * jax.lax.rsqrt(
            jnp.sum(c_all * c_all, axis=1, keepdims=True) + 1e-12
        )
        cn_ref[...] = cf.astype(jnp.bfloat16)

        xt = 2.0 * dc_ref[0] - 1.0
        ts = [jnp.ones_like(xt), xt]
        for _ in range(CHEB_M - 2):
            ts.append(2.0 * xt * ts[-1] - ts[-2])
        phi = jnp.concatenate(ts, axis=0)
        v = jax.lax.dot_general(
            phi, cf, (((1,), (0,)), ((), ())), preferred_element_type=jnp.float32
        )
        vv = jax.lax.dot_general(
            v, v, (((1,), (1,)), ((), ())), preferred_element_type=jnp.float32
        )
        g = jax.lax.dot_general(
            cf, cf, (((0,), (0,)), ((), ())), preferred_element_type=jnp.float32
        )
        s = jnp.sum(phi, axis=1, keepdims=True)
        ss = jax.lax.dot_general(
            s, s, (((1,), (1,)), ((), ())), preferred_element_type=jnp.float32
        )
        dsq_ref[...] += (
            jnp.sum(g * g, axis=(0, 1), keepdims=True)
            - 2.0 * jnp.sum(a_ref[...] * vv, axis=(0, 1), keepdims=True)
            + jnp.sum(a2_ref[...] * ss, axis=(0, 1), keepdims=True)
        )

    row0 = i * BLK
    fn = fn_ref[...]
    cn = cn_ref[...]
    fb = fn_ref[pl.ds(row0, BLK), :]
    cb = cn_ref[pl.ds(row0, BLK), :]

    sim_f = jax.lax.dot_general(
        fb, fn, (((1,), (1,)), ((), ())), preferred_element_type=jnp.float32
    )
    sim_c = jax.lax.dot_general(
        cb, cn, (((1,), (1,)), ((), ())), preferred_element_type=jnp.float32
    )

    cols = jax.lax.broadcasted_iota(jnp.int32, (BLK, n), 1)
    rows = jax.lax.broadcasted_iota(jnp.int32, (BLK, n), 0) + row0
    sim_f = jnp.where(cols == rows, -1e9, sim_f)

    lse = jnp.log(jnp.sum(jnp.exp(sim_c * (1.0 / TEMP)), axis=1, keepdims=True))
    lse_ref[...] += jnp.sum(lse, axis=(0, 1), keepdims=True)

    neg = jnp.float32(-jnp.inf)
    mx = jnp.max(sim_f, axis=1, keepdims=True)
    for _ in range(KNN_K - 1):
        mx = jnp.max(jnp.where(sim_f < mx, sim_f, neg), axis=1, keepdims=True)
    pos_ref[...] += jnp.sum(
        jnp.where(sim_f >= mx, sim_c, 0.0), axis=(0, 1), keepdims=True
    )


def _loss_sums(semantic_codes, dino_features, depth):
    bsz, n, dc = semantic_codes.shape
    df = dino_features.shape[-1]
    depth_c = depth[:, None, :]
    a_w = jnp.asarray(_A_W)
    a_w2 = jnp.asarray(_A_W2)

    grid = (bsz, n // BLK)
    return pl.pallas_call(
        functools.partial(_body, n=n),
        grid=grid,
        in_specs=[
            pl.BlockSpec((1, n, dc), lambda b, i: (b, 0, 0)),
            pl.BlockSpec((1, n, df), lambda b, i: (b, 0, 0)),
            pl.BlockSpec((1, 1, n), lambda b, i: (b, 0, 0)),
            pl.BlockSpec((CHEB_M, CHEB_M), lambda b, i: (0, 0)),
            pl.BlockSpec((CHEB_M, CHEB_M), lambda b, i: (0, 0)),
        ],
        out_specs=[
            pl.BlockSpec((1, 1), lambda b, i: (0, 0)),
            pl.BlockSpec((1, 1), lambda b, i: (0, 0)),
            pl.BlockSpec((1, 1), lambda b, i: (0, 0)),
        ],
        out_shape=[
            jax.ShapeDtypeStruct((1, 1), jnp.float32),
            jax.ShapeDtypeStruct((1, 1), jnp.float32),
            jax.ShapeDtypeStruct((1, 1), jnp.float32),
        ],
        scratch_shapes=[
            pltpu.VMEM((n, dc), jnp.bfloat16),
            pltpu.VMEM((n, df), jnp.bfloat16),
        ],
    )(semantic_codes, dino_features, depth_c, a_w, a_w2)


@jax.jit
def kernel(semantic_codes, dino_features, depth):
    bsz, n, _ = semantic_codes.shape
    pos_sum, lse_sum, dsq_sum = _loss_sums(
        semantic_codes, dino_features, depth
    )

    l_stego = -(pos_sum[0, 0] * (1.0 / TEMP) - KNN_K * lse_sum[0, 0]) / (
        bsz * n * KNN_K
    )
    l_depthg = dsq_sum[0, 0] / (bsz * n * n)
    total = l_stego + LAMBDA_DEPTHG * l_depthg
    return (l_stego, l_depthg, total)


# speedup vs baseline: 1.1651x; 1.0510x over previous
import functools

import jax
import jax.numpy as jnp
import numpy as np
from jax.experimental import pallas as pl
from jax.experimental.pallas import tpu as pltpu

LAMBDA_DEPTHG = 0.3
TEMP = 0.1
KNN_K = 7
SIGMA_D = 0.5

BLK = 2048
CHEB_M = 16


def _cheb_coeff_2d(c, m):
    j = np.arange(m)
    th = np.pi * (j + 0.5) / m
    u = (np.cos(th) + 1.0) / 2.0
    f = np.exp(c * (u[:, None] - u[None, :]) ** 2)
    p = np.cos(np.outer(j, th)) * (2.0 / m)
    p[0] *= 0.5
    return (p @ f @ p.T).astype(np.float32)


_A_W = _cheb_coeff_2d(-1.0 / (2.0 * SIGMA_D * SIGMA_D), CHEB_M)
_A_W2 = _cheb_coeff_2d(-1.0 / (SIGMA_D * SIGMA_D), CHEB_M)


def _body(c_ref, f_ref, dc_ref, a_ref, a2_ref, pos_ref, lse_ref, dsq_ref,
          cn_ref, fn_ref, *, n):
    b = pl.program_id(0)
    i = pl.program_id(1)

    @pl.when((b == 0) & (i == 0))
    def _init():
        pos_ref[...] = jnp.zeros((1, 1), jnp.float32)
        lse_ref[...] = jnp.zeros((1, 1), jnp.float32)
        dsq_ref[...] = jnp.zeros((1, 1), jnp.float32)

    @pl.when(i == 0)
    def _per_batch():
        f_all = f_ref[0]
        fn_ref[...] = (
            f_all * jax.lax.rsqrt(jnp.sum(f_all * f_all, axis=1, keepdims=True) + 1e-12)
        ).astype(jnp.bfloat16)
        c_all = c_ref[0]
        cf = c_all * jax.lax.rsqrt(
            jnp.sum(c_all * c_all, axis=1, keepdims=True) + 1e-12
        )
        cn_ref[...] = cf.astype(jnp.bfloat16)

        xt = 2.0 * dc_ref[0] - 1.0
        ts = [jnp.ones_like(xt), xt]
        for _ in range(CHEB_M - 2):
            ts.append(2.0 * xt * ts[-1] - ts[-2])
        phi = jnp.concatenate(ts, axis=0)
        v = jax.lax.dot_general(
            phi, cf, (((1,), (0,)), ((), ())), preferred_element_type=jnp.float32
        )
        vv = jax.lax.dot_general(
            v, v, (((1,), (1,)), ((), ())), preferred_element_type=jnp.float32
        )
        g = jax.lax.dot_general(
            cf, cf, (((0,), (0,)), ((), ())), preferred_element_type=jnp.float32
        )
        s = jnp.sum(phi, axis=1, keepdims=True)
        ss = jax.lax.dot_general(
            s, s, (((1,), (1,)), ((), ())), preferred_element_type=jnp.float32
        )
        dsq_ref[...] += (
            jnp.sum(g * g, axis=(0, 1), keepdims=True)
            - 2.0 * jnp.sum(a_ref[...] * vv, axis=(0, 1), keepdims=True)
            + jnp.sum(a2_ref[...] * ss, axis=(0, 1), keepdims=True)
        )

    row0 = i * BLK
    fn = fn_ref[...]
    cn = cn_ref[...]
    fb = fn_ref[pl.ds(row0, BLK), :]
    cb = cn_ref[pl.ds(row0, BLK), :]

    sim_f = jax.lax.dot_general(
        fb, fn, (((1,), (1,)), ((), ())), preferred_element_type=jnp.float32
    )
    sim_c = jax.lax.dot_general(
        cb, cn, (((1,), (1,)), ((), ())), preferred_element_type=jnp.float32
    )

    cols = jax.lax.broadcasted_iota(jnp.int32, (BLK, n), 1)
    rows = jax.lax.broadcasted_iota(jnp.int32, (BLK, n), 0) + row0
    sim_f = jnp.where(cols == rows, -1e9, sim_f)

    lse = jnp.log(jnp.sum(jnp.exp(sim_c * (1.0 / TEMP)), axis=1, keepdims=True))
    lse_ref[...] += jnp.sum(lse, axis=(0, 1), keepdims=True)

    neg = jnp.float32(-jnp.inf)
    mx = jnp.max(sim_f, axis=1, keepdims=True)
    for _ in range(KNN_K - 1):
        mx = jnp.max(jnp.where(sim_f < mx, sim_f, neg), axis=1, keepdims=True)
    pos_ref[...] += jnp.sum(
        jnp.where(sim_f >= mx, sim_c, 0.0), axis=(0, 1), keepdims=True
    )


def _loss_sums(semantic_codes, dino_features, depth):
    bsz, n, dc = semantic_codes.shape
    df = dino_features.shape[-1]
    depth_c = depth[:, None, :]
    a_w = jnp.asarray(_A_W)
    a_w2 = jnp.asarray(_A_W2)

    grid = (bsz, n // BLK)
    return pl.pallas_call(
        functools.partial(_body, n=n),
        grid=grid,
        in_specs=[
            pl.BlockSpec((1, n, dc), lambda b, i: (b, 0, 0)),
            pl.BlockSpec((1, n, df), lambda b, i: (b, 0, 0)),
            pl.BlockSpec((1, 1, n), lambda b, i: (b, 0, 0)),
            pl.BlockSpec((CHEB_M, CHEB_M), lambda b, i: (0, 0)),
            pl.BlockSpec((CHEB_M, CHEB_M), lambda b, i: (0, 0)),
        ],
        out_specs=[
            pl.BlockSpec((1, 1), lambda b, i: (0, 0)),
            pl.BlockSpec((1, 1), lambda b, i: (0, 0)),
            pl.BlockSpec((1, 1), lambda b, i: (0, 0)),
        ],
        out_shape=[
            jax.ShapeDtypeStruct((1, 1), jnp.float32),
            jax.ShapeDtypeStruct((1, 1), jnp.float32),
            jax.ShapeDtypeStruct((1, 1), jnp.float32),
        ],
        scratch_shapes=[
            pltpu.VMEM((n, dc), jnp.bfloat16),
            pltpu.VMEM((n, df), jnp.bfloat16),
        ],
    )(semantic_codes, dino_features, depth_c, a_w, a_w2)


@jax.jit
def kernel(semantic_codes, dino_features, depth):
    bsz, n, _ = semantic_codes.shape
    pos_sum, lse_sum, dsq_sum = _loss_sums(
        semantic_codes, dino_features, depth
    )

    l_stego = -(pos_sum[0, 0] * (1.0 / TEMP) - KNN_K * lse_sum[0, 0]) / (
        bsz * n * KNN_K
    )
    l_depthg = dsq_sum[0, 0] / (bsz * n * n)
    total = l_stego + LAMBDA_DEPTHG * l_depthg
    return (l_stego, l_depthg, total)
